# baseline (speedup 1.0000x reference)
import functools

import jax
import jax.numpy as jnp
from jax import lax
from jax.experimental import pallas as pl
from jax.experimental.pallas import tpu as pltpu
from jax.experimental.pallas import tpu_sc as plsc

VOCAB = 1000
NS_TOK = 50
BATCH = 1024
NC, NS = 2, 16
VN = 256
NITEM = NS_TOK


@functools.partial(
    pl.kernel,
    out_type=jax.ShapeDtypeStruct((NS_TOK, VOCAB, BATCH), jnp.float32),
    mesh=plsc.VectorSubcoreMesh(
        core_axis_name="c", subcore_axis_name="s", num_cores=NC, num_subcores=NS
    ),
    scratch_types=[
        pltpu.VMEM((NS_TOK, 128), jnp.int32),
        pltpu.VMEM((VN, 128), jnp.float32),
        pltpu.VMEM((VN, 128), jnp.float32),
        pltpu.VMEM((VN, 128), jnp.float32),
        pltpu.SemaphoreType.DMA,
        pltpu.SemaphoreType.DMA,
        pltpu.SemaphoreType.DMA,
    ],
    compiler_params=pltpu.CompilerParams(
        needs_layout_passes=False,
        use_tc_tiling_on_sc=True,
        disable_bounds_checks=True,
        disable_semaphore_checks=True,
        skip_device_barrier=True,
    ),
)
def _one_hot_sc(tok_hbm, out_hbm, tok_v, buf0, buf1, buf2, sem0, sem1, sem2):
    wid = lax.axis_index("s") * NC + lax.axis_index("c")
    tb = wid >> 2
    c = wid & 3
    v0 = jnp.where(c == 3, VOCAB - VN, c * VN)
    bufs = (buf0, buf1, buf2)
    sems = (sem0, sem1, sem2)

    pltpu.sync_copy(tok_hbm.at[:, pl.ds(tb * 128, 128)], tok_v)

    zeros = jnp.zeros((16,), jnp.float32)
    ones = jnp.ones((16,), jnp.float32)
    lane = lax.broadcasted_iota(jnp.int32, (16,), 0)

    def _memset(buf):
        def _row(i, carry):
            for k in range(8):
                buf[i, pl.ds(k * 16, 16)] = zeros
            return carry

        lax.fori_loop(0, VN, _row, None)

    def _scatter(buf, s, vals):
        def _group(g, carry):
            tv = tok_v[s, pl.ds(g * 16, 16)]
            mask = (tv >= v0) & (tv < v0 + VN)
            plsc.store_scatter(buf, [tv - v0, lane + g * 16], vals, mask=mask)
            return carry

        lax.fori_loop(0, 8, _group, None)

    def _dma_out(b, s):
        dst = out_hbm.at[s, pl.ds(v0, VN), pl.ds(tb * 128, 128)]
        pltpu.async_copy(bufs[b], dst, sems[b])

    def _dma_wait(b, s):
        dst = out_hbm.at[s, pl.ds(v0, VN), pl.ds(tb * 128, 128)]
        pltpu.make_async_copy(bufs[b], dst, sems[b]).wait()

    for b in range(3):
        _memset(bufs[b])
        _scatter(bufs[b], jnp.int32(b), ones)
        _dma_out(b, jnp.int32(b))

    def _step(i, carry):
        for b in range(3):
            s = i * 3 + b
            _dma_wait(b, s - 3)
            _scatter(bufs[b], s - 3, zeros)
            _scatter(bufs[b], s, ones)
            _dma_out(b, s)
        return carry

    lax.fori_loop(1, (NITEM - 2) // 3, _step, None)

    for b, s in ((0, NITEM - 2), (1, NITEM - 1)):
        _dma_wait(b, jnp.int32(s - 3))
        _scatter(bufs[b], jnp.int32(s - 3), zeros)
        _scatter(bufs[b], jnp.int32(s), ones)
        _dma_out(b, jnp.int32(s))

    for b, s in ((2, NITEM - 3), (0, NITEM - 2), (1, NITEM - 1)):
        _dma_wait(b, jnp.int32(s))


def kernel(tokens):
    tok_t = jnp.swapaxes(tokens.astype(jnp.int32), 0, 1)
    out = _one_hot_sc(tok_t)
    return jnp.transpose(out, (2, 0, 1))


# speedup vs baseline: 1.0106x; 1.0106x over previous
import functools

import jax
import jax.numpy as jnp
from jax import lax
from jax.experimental import pallas as pl
from jax.experimental.pallas import tpu as pltpu
from jax.experimental.pallas import tpu_sc as plsc

VOCAB = 1000
NS_TOK = 50
BATCH = 1024
NC, NS = 2, 16
VN = 256
NITEM = NS_TOK


@functools.partial(
    pl.kernel,
    out_type=jax.ShapeDtypeStruct((NS_TOK, VOCAB, BATCH), jnp.float32),
    mesh=plsc.VectorSubcoreMesh(
        core_axis_name="c", subcore_axis_name="s", num_cores=NC, num_subcores=NS
    ),
    scratch_types=[
        pltpu.VMEM((NS_TOK, 128), jnp.int32),
        pltpu.VMEM((VN, 128), jnp.float32),
        pltpu.VMEM((VN, 128), jnp.float32),
        pltpu.SemaphoreType.DMA,
        pltpu.SemaphoreType.DMA,
    ],
    compiler_params=pltpu.CompilerParams(
        needs_layout_passes=False,
        use_tc_tiling_on_sc=True,
        disable_bounds_checks=True,
        disable_semaphore_checks=True,
        skip_device_barrier=True,
    ),
)
def _one_hot_sc(tok_hbm, out_hbm, tok_v, buf0, buf1, sem0, sem1):
    wid = lax.axis_index("s") * NC + lax.axis_index("c")
    tb = wid >> 2
    c = wid & 3
    v0 = jnp.where(c == 3, VOCAB - VN, c * VN)
    bufs = (buf0, buf1)
    sems = (sem0, sem1)

    pltpu.sync_copy(tok_hbm.at[:, pl.ds(tb * 128, 128)], tok_v)

    zeros = jnp.zeros((16,), jnp.float32)
    ones = jnp.ones((16,), jnp.float32)
    lane = lax.broadcasted_iota(jnp.int32, (16,), 0)

    def _memset(buf):
        def _row(i, carry):
            for k in range(8):
                buf[i, pl.ds(k * 16, 16)] = zeros
            return carry

        lax.fori_loop(0, VN, _row, None)

    def _scatter(buf, s, vals):
        def _group(g, carry):
            tv = tok_v[s, pl.ds(g * 16, 16)]
            mask = (tv >= v0) & (tv < v0 + VN)
            plsc.store_scatter(buf, [tv - v0, lane + g * 16], vals, mask=mask)
            return carry

        lax.fori_loop(0, 8, _group, None)

    def _dma_out(b, s):
        dst = out_hbm.at[s, pl.ds(v0, VN), pl.ds(tb * 128, 128)]
        pltpu.async_copy(bufs[b], dst, sems[b])

    def _dma_wait(b, s):
        dst = out_hbm.at[s, pl.ds(v0, VN), pl.ds(tb * 128, 128)]
        pltpu.make_async_copy(bufs[b], dst, sems[b]).wait()

    for b in range(2):
        _memset(bufs[b])
        _scatter(bufs[b], jnp.int32(b), ones)
        _dma_out(b, jnp.int32(b))

    def _step(i, carry):
        for b in range(2):
            s = i * 2 + b
            _dma_wait(b, s - 2)
            _scatter(bufs[b], s - 2, zeros)
            _scatter(bufs[b], s, ones)
            _dma_out(b, s)
        return carry

    lax.fori_loop(1, NITEM // 2, _step, None)

    for b in range(2):
        _dma_wait(b, jnp.int32(NITEM - 2 + b))


def kernel(tokens):
    tok_t = jnp.swapaxes(tokens.astype(jnp.int32), 0, 1)
    out = _one_hot_sc(tok_t)
    return jnp.transpose(out, (2, 0, 1))


# speedup vs baseline: 1.0499x; 1.0389x over previous
import functools

import jax
import jax.numpy as jnp
from jax import lax
from jax.experimental import pallas as pl
from jax.experimental.pallas import tpu as pltpu
from jax.experimental.pallas import tpu_sc as plsc

VOCAB = 1000
NS_TOK = 50
BATCH = 1024
NC, NS = 2, 16
VN = 256
NITEM = NS_TOK


@functools.partial(
    pl.kernel,
    out_type=jax.ShapeDtypeStruct((NS_TOK, VOCAB, BATCH), jnp.float32),
    mesh=plsc.VectorSubcoreMesh(
        core_axis_name="c", subcore_axis_name="s", num_cores=NC, num_subcores=NS
    ),
    scratch_types=[
        pltpu.VMEM((NS_TOK, 128), jnp.int32),
        pltpu.VMEM((VN, 128), jnp.float32),
        pltpu.VMEM((VN, 128), jnp.float32),
        pltpu.SemaphoreType.DMA,
        pltpu.SemaphoreType.DMA,
    ],
    compiler_params=pltpu.CompilerParams(
        needs_layout_passes=False,
        use_tc_tiling_on_sc=True,
        disable_bounds_checks=True,
        disable_semaphore_checks=True,
        skip_device_barrier=True,
    ),
)
def _one_hot_sc(tok_hbm, out_hbm, tok_v, buf0, buf1, sem0, sem1):
    wid = lax.axis_index("s") * NC + lax.axis_index("c")
    tb = wid >> 2
    c = wid & 3
    v0 = jnp.where(c == 3, VOCAB - VN, c * VN)
    bufs = (buf0, buf1)
    sems = (sem0, sem1)

    tok_copy = pltpu.async_copy(tok_hbm.at[:, pl.ds(tb * 128, 128)], tok_v, sem0)

    zeros = jnp.zeros((16,), jnp.float32)
    ones = jnp.ones((16,), jnp.float32)
    lane = lax.broadcasted_iota(jnp.int32, (16,), 0)

    def _memset(buf):
        def _rows(i, carry):
            for d in range(2):
                for k in range(8):
                    buf[i * 2 + d, pl.ds(k * 16, 16)] = zeros
            return carry

        lax.fori_loop(0, VN // 2, _rows, None)

    def _scatter(buf, s, vals):
        def _group(g, carry):
            tv = tok_v[s, pl.ds(g * 16, 16)]
            mask = (tv >= v0) & (tv < v0 + VN)
            plsc.store_scatter(buf, [tv - v0, lane + g * 16], vals, mask=mask)
            return carry

        lax.fori_loop(0, 8, _group, None)

    def _dma_out(b, s):
        dst = out_hbm.at[s, pl.ds(v0, VN), pl.ds(tb * 128, 128)]
        pltpu.async_copy(bufs[b], dst, sems[b])

    def _dma_wait(b, s):
        dst = out_hbm.at[s, pl.ds(v0, VN), pl.ds(tb * 128, 128)]
        pltpu.make_async_copy(bufs[b], dst, sems[b]).wait()

    _memset(bufs[0])
    tok_copy.wait()
    _scatter(bufs[0], jnp.int32(0), ones)
    _dma_out(0, jnp.int32(0))
    _memset(bufs[1])
    _scatter(bufs[1], jnp.int32(1), ones)
    _dma_out(1, jnp.int32(1))

    def _step(i, carry):
        for b in range(2):
            s = i * 2 + b
            _dma_wait(b, s - 2)
            _scatter(bufs[b], s - 2, zeros)
            _scatter(bufs[b], s, ones)
            _dma_out(b, s)
        return carry

    lax.fori_loop(1, NITEM // 2, _step, None)

    for b in range(2):
        _dma_wait(b, jnp.int32(NITEM - 2 + b))


def kernel(tokens):
    tok_t = jnp.swapaxes(tokens.astype(jnp.int32), 0, 1)
    out = _one_hot_sc(tok_t)
    return jnp.transpose(out, (2, 0, 1))
